# baseline (speedup 1.0000x reference)
import jax
import jax.numpy as jnp
from jax.experimental import pallas as pl
from jax.experimental.pallas import tpu as pltpu

_E = 320000
_EMB = 128
_OUT = 256
_NR = 6
_N_SEG = 10000.0
_BE = 16000
_HALF = _E // (2 * _BE)


def _ppblock_kernel(rbfT_ref, ma_ref, mb_ref, wrbfT_ref, wup_ref, wd0_ref,
                    wd1_ref, wd2_ref, bias_ref, out_ref, acc_ref):
    i = pl.program_id(0)

    @pl.when(i == 0)
    def _init():
        acc_ref[...] = jnp.zeros_like(acc_ref)

    acc_ref[...] += (
        jnp.dot(rbfT_ref[:, pl.ds(i * _BE, _BE)], ma_ref[...],
                preferred_element_type=jnp.float32)
        + jnp.dot(rbfT_ref[:, pl.ds((i + _HALF) * _BE, _BE)], mb_ref[...],
                  preferred_element_type=jnp.float32))

    @pl.when(i == pl.num_programs(0) - 1)
    def _finish():
        s = jnp.sum(wrbfT_ref[...] * acc_ref[...], axis=0, keepdims=True)
        dn = (((1,), (1,)), ((), ()))
        v = jax.lax.dot_general(s, wup_ref[...], dn,
                                preferred_element_type=jnp.float32)
        v = jax.lax.dot_general(v, wd0_ref[...], dn,
                                preferred_element_type=jnp.float32)
        v += _N_SEG * bias_ref[0:1, :]
        v = jax.lax.dot_general(v, wd1_ref[...], dn,
                                preferred_element_type=jnp.float32)
        v += _N_SEG * bias_ref[1:2, :]
        v = jax.lax.dot_general(v, wd2_ref[...], dn,
                                preferred_element_type=jnp.float32)
        v += _N_SEG * bias_ref[2:3, :]
        out_ref[...] = v


def kernel(m, rbf, edge_index, W_rbf, W_up, W_d0, b_d0, W_d1, b_d1, W_d2, b_d2):
    del edge_index
    rbfT = rbf.T
    wrbfT = W_rbf.T
    bias = jnp.stack([b_d0, b_d1, b_d2])
    return pl.pallas_call(
        _ppblock_kernel,
        grid=(_HALF,),
        in_specs=[
            pl.BlockSpec((_NR, _E), lambda i: (0, 0)),
            pl.BlockSpec((_BE, _EMB), lambda i: (i, 0)),
            pl.BlockSpec((_BE, _EMB), lambda i: (i + _HALF, 0)),
            pl.BlockSpec((_NR, _EMB), lambda i: (0, 0)),
            pl.BlockSpec((_OUT, _EMB), lambda i: (0, 0)),
            pl.BlockSpec((_OUT, _OUT), lambda i: (0, 0)),
            pl.BlockSpec((_OUT, _OUT), lambda i: (0, 0)),
            pl.BlockSpec((_OUT, _OUT), lambda i: (0, 0)),
            pl.BlockSpec((3, _OUT), lambda i: (0, 0)),
        ],
        out_specs=pl.BlockSpec((1, _OUT), lambda i: (0, 0)),
        out_shape=jax.ShapeDtypeStruct((1, _OUT), jnp.float32),
        scratch_shapes=[pltpu.VMEM((_NR, _EMB), jnp.float32)],
        compiler_params=pltpu.CompilerParams(
            allow_input_fusion=[True, False, False, False, False, False,
                                False, False, False]),
    )(rbfT, m, m, wrbfT, W_up, W_d0, W_d1, W_d2, bias)


# speedup vs baseline: 1.0283x; 1.0283x over previous
import jax
import jax.numpy as jnp
from jax.experimental import pallas as pl
from jax.experimental.pallas import tpu as pltpu

_E = 320000
_EMB = 128
_OUT = 256
_NR = 6
_N_SEG = 10000.0
_BE = 16000
_STEPS = _E // _BE


def _ppblock_kernel(rbfT_ref, m_ref, wrbfT_ref, wup_ref, wd0_ref,
                    wd1_ref, wd2_ref, bias_ref, out_ref, acc_ref):
    i = pl.program_id(0)

    @pl.when(i == 0)
    def _init():
        acc_ref[...] = jnp.zeros_like(acc_ref)

    acc_ref[...] += jnp.dot(rbfT_ref[:, pl.ds(i * _BE, _BE)], m_ref[...],
                            preferred_element_type=jnp.float32)

    @pl.when(i == pl.num_programs(0) - 1)
    def _finish():
        s = jnp.sum(wrbfT_ref[...] * acc_ref[...], axis=0, keepdims=True)
        dn = (((1,), (1,)), ((), ()))
        v = jax.lax.dot_general(s, wup_ref[...], dn,
                                preferred_element_type=jnp.float32)
        v = jax.lax.dot_general(v, wd0_ref[...], dn,
                                preferred_element_type=jnp.float32)
        v += _N_SEG * bias_ref[0:1, :]
        v = jax.lax.dot_general(v, wd1_ref[...], dn,
                                preferred_element_type=jnp.float32)
        v += _N_SEG * bias_ref[1:2, :]
        v = jax.lax.dot_general(v, wd2_ref[...], dn,
                                preferred_element_type=jnp.float32)
        v += _N_SEG * bias_ref[2:3, :]
        out_ref[...] = v


def kernel(m, rbf, edge_index, W_rbf, W_up, W_d0, b_d0, W_d1, b_d1, W_d2, b_d2):
    del edge_index
    rbfT = rbf.T
    wrbfT = W_rbf.T
    bias = jnp.stack([b_d0, b_d1, b_d2])
    return pl.pallas_call(
        _ppblock_kernel,
        grid=(_STEPS,),
        in_specs=[
            pl.BlockSpec((_NR, _E), lambda i: (0, 0)),
            pl.BlockSpec((_BE, _EMB), lambda i: (i, 0)),
            pl.BlockSpec((_NR, _EMB), lambda i: (0, 0)),
            pl.BlockSpec((_OUT, _EMB), lambda i: (0, 0)),
            pl.BlockSpec((_OUT, _OUT), lambda i: (0, 0)),
            pl.BlockSpec((_OUT, _OUT), lambda i: (0, 0)),
            pl.BlockSpec((_OUT, _OUT), lambda i: (0, 0)),
            pl.BlockSpec((3, _OUT), lambda i: (0, 0)),
        ],
        out_specs=pl.BlockSpec((1, _OUT), lambda i: (0, 0)),
        out_shape=jax.ShapeDtypeStruct((1, _OUT), jnp.float32),
        scratch_shapes=[pltpu.VMEM((_NR, _EMB), jnp.float32)],
        compiler_params=pltpu.CompilerParams(
            allow_input_fusion=[True, False, False, False, False, False,
                                False, False]),
    )(rbfT, m, wrbfT, W_up, W_d0, W_d1, W_d2, bias)
